# Initial kernel scaffold; baseline (speedup 1.0000x reference)
import jax
import jax.numpy as jnp
from jax.experimental import pallas as pl


def kernel(inputs):
    raise NotImplementedError("write your pallas kernel here")


# speedup vs baseline: 2.3770x; 2.3770x over previous
import jax
import jax.numpy as jnp
from jax.experimental import pallas as pl
from jax.experimental.pallas import tpu as pltpu

_HF = 256
_HO = _HF * 4


def _split_bf16(x):
    hi = x.astype(jnp.bfloat16)
    lo = (x - hi.astype(jnp.float32)).astype(jnp.bfloat16)
    return hi, lo


def _rf_body(v_ref, ew_ref, eh_ref, o_ref):
    v = jnp.maximum(v_ref[0], 0.0)
    vh, vl = _split_bf16(v)
    ew = ew_ref[...]
    u = jnp.dot(vh, ew, preferred_element_type=jnp.float32)
    u = u + jnp.dot(vl, ew, preferred_element_type=jnp.float32)
    ul = jnp.concatenate([u[:, :4], u[:, :-4]], axis=1)
    ur = jnp.concatenate([u[:, 4:], u[:, -4:]], axis=1)
    r = jax.lax.broadcasted_iota(jnp.int32, (_HF, _HO), 1) & 3
    ow = jnp.maximum(jnp.maximum(u, jnp.where(r <= 2, ul, u)),
                     jnp.where(r >= 1, ur, u))
    oh, ol = _split_bf16(ow)
    eh = eh_ref[...]
    w = jnp.dot(eh, oh, preferred_element_type=jnp.float32)
    w = w + jnp.dot(eh, ol, preferred_element_type=jnp.float32)
    wu = jnp.concatenate([w[:4], w[:-4]], axis=0)
    wd = jnp.concatenate([w[4:], w[-4:]], axis=0)
    rr = jax.lax.broadcasted_iota(jnp.int32, (_HO, _HO), 0) & 3
    out = jnp.maximum(jnp.maximum(w, jnp.where(rr <= 2, wu, w)),
                      jnp.where(rr >= 1, wd, w))
    o_ref[0] = out


def kernel(inputs):
    b = inputs.shape[0]
    nmaps = b * 3
    v = jnp.transpose(inputs[..., :3], (0, 3, 1, 2)).reshape(nmaps, _HF, _HF)
    i = jnp.arange(_HF, dtype=jnp.int32)
    p = jnp.arange(_HO, dtype=jnp.int32)
    ew = (i[:, None] == (p[None, :] // 4)).astype(jnp.bfloat16)
    ehm = ((p[:, None] // 4) == i[None, :]).astype(jnp.bfloat16)

    out = pl.pallas_call(
        _rf_body,
        grid=(nmaps,),
        in_specs=[
            pl.BlockSpec((1, _HF, _HF), lambda m: (m, 0, 0)),
            pl.BlockSpec((_HF, _HO), lambda m: (0, 0)),
            pl.BlockSpec((_HO, _HF), lambda m: (0, 0)),
        ],
        out_specs=pl.BlockSpec((1, _HO, _HO), lambda m: (m, 0, 0)),
        out_shape=jax.ShapeDtypeStruct((nmaps, _HO, _HO), jnp.float32),
        compiler_params=pltpu.CompilerParams(
            dimension_semantics=("parallel",),
        ),
    )(v, ew, ehm)
    return out.reshape(b, 3, _HO, _HO)


# speedup vs baseline: 2.7067x; 1.1387x over previous
import jax
import jax.numpy as jnp
from jax.experimental import pallas as pl
from jax.experimental.pallas import tpu as pltpu

_HF = 256
_HO = _HF * 4


_G = 4


def _rf_body(v_ref, eh_ref, e768_ref, mhl_ref, mhr_ref, o_ref):
    mhl = jnp.concatenate([mhl_ref[...], mhl_ref[...]], axis=1)
    mhr = jnp.concatenate([mhr_ref[...], mhr_ref[...]], axis=1)
    for g in range(_G):
        v = jnp.maximum(v_ref[g], 0.0).astype(jnp.bfloat16)
        u = jnp.dot(eh_ref[...], v, preferred_element_type=jnp.float32)
        uu = jnp.concatenate([u[:4], u[:-4]], axis=0)
        ud = jnp.concatenate([u[4:], u[-4:]], axis=0)
        oh = jnp.maximum(jnp.maximum(u, uu * mhl), ud * mhr)
        left = jnp.concatenate([oh[:, :1], oh[:, :-1]], axis=1)
        right = jnp.concatenate([oh[:, 1:], oh[:, -1:]], axis=1)
        a = jnp.maximum(left, oh)
        b = jnp.maximum(oh, right)
        c = jnp.maximum(a, right)
        lhs = jnp.concatenate([a, c, b], axis=1).astype(jnp.bfloat16)
        o_ref[g] = jnp.dot(lhs, e768_ref[...],
                           preferred_element_type=jnp.float32)


def kernel(inputs):
    bsz = inputs.shape[0]
    nmaps = bsz * 3
    v = jnp.transpose(inputs[..., :3], (0, 3, 1, 2)).reshape(nmaps, _HF, _HF)
    i = jnp.arange(_HF, dtype=jnp.int32)
    p = jnp.arange(_HO, dtype=jnp.int32)
    q, r = p // 4, p % 4
    ehm = (q[:, None] == i[None, :]).astype(jnp.bfloat16)
    src = jnp.where(r == 0, q, jnp.where(r == 3, 512 + q, 256 + q))
    e768 = (jnp.arange(768, dtype=jnp.int32)[:, None] == src[None, :]
            ).astype(jnp.bfloat16)
    mhl = jnp.tile((r <= 2).astype(jnp.float32)[:, None], (1, 128))
    mhr = jnp.tile((r >= 1).astype(jnp.float32)[:, None], (1, 128))

    out = pl.pallas_call(
        _rf_body,
        grid=(nmaps // _G,),
        in_specs=[
            pl.BlockSpec((_G, _HF, _HF), lambda m: (m, 0, 0)),
            pl.BlockSpec((_HO, _HF), lambda m: (0, 0)),
            pl.BlockSpec((768, _HO), lambda m: (0, 0)),
            pl.BlockSpec((_HO, 128), lambda m: (0, 0)),
            pl.BlockSpec((_HO, 128), lambda m: (0, 0)),
        ],
        out_specs=pl.BlockSpec((_G, _HO, _HO), lambda m: (m, 0, 0)),
        out_shape=jax.ShapeDtypeStruct((nmaps, _HO, _HO), jnp.float32),
        compiler_params=pltpu.CompilerParams(
            dimension_semantics=("parallel",),
        ),
    )(v, ehm, e768, mhl, mhr)
    return out.reshape(bsz, 3, _HO, _HO)
